# baseline (speedup 1.0000x reference)
import functools

import jax
import jax.numpy as jnp
from jax import lax
from jax.experimental import pallas as pl
from jax.experimental.pallas import tpu as pltpu
from jax.experimental.pallas import tpu_sc as plsc

N = 10000
E = 160000
NP = 10240
H = 512
OUT = 64
BR = 1024

NC = 2
NS = 16
L = 16
RC = 2000
KSUB = 128
KEPT = RC + 176
HALF = NP // NC
ACC_SUB = 10368
RPT = ACC_SUB // NS
NCH = E // RC


def _make_sc_agg(D):
    S = D // 128
    KR = KSUB // S
    chunk_rows = 2560 if D == 512 else 5120
    qpc = HALF // chunk_rows
    own = chunk_rows // NS
    real_sub = chunk_rows * S
    wpt = real_sub // NS

    mesh = plsc.VectorSubcoreMesh(core_axis_name="c", subcore_axis_name="s")

    @functools.partial(
        pl.kernel,
        out_type=jax.ShapeDtypeStruct((NP * S, 128), jnp.float32),
        mesh=mesh,
        scratch_types=[
            pltpu.VMEM((RC,), jnp.int32),
            pltpu.VMEM((RC,), jnp.int32),
            pltpu.VMEM((RC,), jnp.int32),
            pltpu.VMEM((RC,), jnp.int32),
            pltpu.VMEM((KEPT,), jnp.int32),
            pltpu.VMEM((KEPT,), jnp.int32),
            pltpu.VMEM((KSUB,), jnp.int32),
            pltpu.VMEM((KSUB,), jnp.int32),
            pltpu.VMEM((KSUB,), jnp.int32),
            pltpu.VMEM((KSUB,), jnp.int32),
            pltpu.VMEM((KSUB, 128), jnp.float32),
            pltpu.VMEM((KSUB, 128), jnp.float32),
            pltpu.VMEM_SHARED((ACC_SUB, 128), jnp.float32),
            pltpu.SemaphoreType.DMA,
            pltpu.SemaphoreType.DMA,
            pltpu.SemaphoreType.DMA,
            pltpu.SemaphoreType.DMA,
            pltpu.SemaphoreType.DMA,
        ],
        compiler_params=pltpu.CompilerParams(needs_layout_passes=False),
    )
    def body(h_hbm, src_hbm, dst_hbm, zeros_hbm, out_hbm,
             raw_srcA, raw_dstA, raw_srcB, raw_dstB,
             kept_src, kept_dst, idxgA, idxdA, idxgB, idxdB, gbufA, gbufB,
             acc, sem, semA, semB, semC, semD):
        c = lax.axis_index("c")
        s = lax.axis_index("s")
        lane = lax.iota(jnp.int32, L)
        dummy = jnp.full((L,), chunk_rows, jnp.int32)
        zi = jnp.zeros((L,), jnp.int32)

        def build_idx(o, idxg, idxd):
            for hh in range(KR // L):
                sv = kept_src[pl.ds(o + hh * L, L)]
                dv = kept_dst[pl.ds(o + hh * L, L)]
                for t in range(S):
                    idxg[pl.ds(t * KR + hh * L, L)] = sv * S + t
                    idxd[pl.ds(t * KR + hh * L, L)] = dv * S + t

        def drain2(npair):
            def pbody(q, carry):
                build_idx(2 * q * KR, idxgA, idxdA)
                cpA = pltpu.async_copy(h_hbm.at[idxgA], gbufA, sem)
                build_idx((2 * q + 1) * KR, idxgB, idxdB)
                cpB = pltpu.async_copy(h_hbm.at[idxgB], gbufB, semB)
                cpA.wait()
                pltpu.sync_copy(gbufA, acc.at[idxdA], add=True)
                cpB.wait()
                pltpu.sync_copy(gbufB, acc.at[idxdB], add=True)
                return carry
            lax.fori_loop(0, npair, pbody, 0)

        for qi in range(qpc):
            lo = (c * qpc + qi) * chunk_rows
            tlo = lo + s * own

            pltpu.sync_copy(zeros_hbm, acc.at[pl.ds(s * RPT, RPT)])
            plsc.subcore_barrier()

            def filt(raw_src, raw_dst, offv):
                def fbody(i, offv):
                    sv = raw_src[pl.ds(i * L, L)]
                    dv = raw_dst[pl.ds(i * L, L)]
                    m = (dv >= tlo) & (dv < tlo + own)
                    mi = m.astype(jnp.int32)
                    nav = plsc.all_reduce_population_count(m)
                    pos = offv + plsc.cumsum(mi) - 1
                    plsc.store_scatter(kept_src, [pos], sv, mask=m)
                    plsc.store_scatter(kept_dst, [pos], dv - lo, mask=m)
                    return offv + nav
                return lax.fori_loop(0, RC // L, fbody, offv)

            def drain_move(offv):
                cnt = offv[0]
                nbp = cnt // (2 * KR)
                drain2(nbp)
                o = nbp * 2 * KR
                for t in range(2 * KR // L):
                    vs = kept_src[pl.ds(o + t * L, L)]
                    vd = kept_dst[pl.ds(o + t * L, L)]
                    kept_src[pl.ds(t * L, L)] = vs
                    kept_dst[pl.ds(t * L, L)] = vd
                return offv - nbp * (2 * KR)

            def rpair(p, offv):
                i0 = 2 * p
                i1 = 2 * p + 1
                c0s = pltpu.async_copy(src_hbm.at[pl.ds(i0 * RC, RC)],
                                       raw_srcA, semA)
                c0d = pltpu.async_copy(dst_hbm.at[pl.ds(i0 * RC, RC)],
                                       raw_dstA, semB)
                c1s = pltpu.async_copy(src_hbm.at[pl.ds(i1 * RC, RC)],
                                       raw_srcB, semC)
                c1d = pltpu.async_copy(dst_hbm.at[pl.ds(i1 * RC, RC)],
                                       raw_dstB, semD)
                c0s.wait()
                c0d.wait()
                offv = drain_move(filt(raw_srcA, raw_dstA, offv))
                c1s.wait()
                c1d.wait()
                return drain_move(filt(raw_srcB, raw_dstB, offv))
            remv = lax.fori_loop(0, NCH // 2, rpair, zi)

            rem = remv[0]
            for t in range(2 * KR // L):
                plsc.store_scatter(kept_src, [remv + t * L + lane], zi)
                plsc.store_scatter(kept_dst, [remv + t * L + lane], dummy)
            drain2((rem + 2 * KR - 1) // (2 * KR))

            pltpu.sync_copy(
                acc.at[pl.ds(s * wpt, wpt)],
                out_hbm.at[pl.ds((c * qpc + qi) * real_sub + s * wpt, wpt)])
            plsc.subcore_barrier()

    return body


_sc_agg_256 = _make_sc_agg(256)
_sc_agg_512 = _make_sc_agg(512)


def _dense_body(h_ref, agg_ref, wlt, bl, wrt, w1t, b1, w2t, b2, out_ref):
    t = (
        jnp.dot(agg_ref[...], wlt[...], preferred_element_type=jnp.float32)
        + bl[...]
        + jnp.dot(h_ref[...], wrt[...], preferred_element_type=jnp.float32)
    )
    h1 = jnp.maximum(
        jnp.dot(t, w1t[...], preferred_element_type=jnp.float32) + b1[...], 0.0
    )
    h2 = jnp.maximum(
        jnp.dot(h1, w2t[...], preferred_element_type=jnp.float32) + b2[...], 0.0
    )
    out_ref[...] = h2


def _final_body(h_ref, agg_ref, wlt, bl, wrt, w1t, b1, w2t, b2,
                fc1t, fc1b, fc2t, fc2b, out_ref):
    t = (
        jnp.dot(agg_ref[...], wlt[...], preferred_element_type=jnp.float32)
        + bl[...]
        + jnp.dot(h_ref[...], wrt[...], preferred_element_type=jnp.float32)
    )
    h1 = jnp.maximum(
        jnp.dot(t, w1t[...], preferred_element_type=jnp.float32) + b1[...], 0.0
    )
    h2 = jnp.maximum(
        jnp.dot(h1, w2t[...], preferred_element_type=jnp.float32) + b2[...], 0.0
    )
    f1 = jnp.maximum(
        jnp.dot(h2, fc1t[...], preferred_element_type=jnp.float32) + fc1b[...], 0.0
    )
    f2 = jnp.dot(f1, fc2t[...], preferred_element_type=jnp.float32) + fc2b[...]
    out_ref[...] = 1.0 / (1.0 + jnp.exp(-f2))


def _wspec(shape):
    return pl.BlockSpec(shape, lambda i: (0, 0))


def _dense_layer(h, agg, wlt, bl, wrt, w1t, b1, w2t, b2):
    din = h.shape[1]
    grid = (NP // BR,)
    return pl.pallas_call(
        _dense_body,
        grid=grid,
        in_specs=[
            pl.BlockSpec((BR, din), lambda i: (i, 0)),
            pl.BlockSpec((BR, din), lambda i: (i, 0)),
            _wspec((din, H)), _wspec((1, H)), _wspec((din, H)),
            _wspec((H, H)), _wspec((1, H)), _wspec((H, H)), _wspec((1, H)),
        ],
        out_specs=pl.BlockSpec((BR, H), lambda i: (i, 0)),
        out_shape=jax.ShapeDtypeStruct((NP, H), jnp.float32),
    )(h, agg, wlt, bl, wrt, w1t, b1, w2t, b2)


def _final_layer(h, agg, wlt, bl, wrt, w1t, b1, w2t, b2, fc1t, fc1b, fc2t, fc2b):
    din = h.shape[1]
    grid = (NP // BR,)
    return pl.pallas_call(
        _final_body,
        grid=grid,
        in_specs=[
            pl.BlockSpec((BR, din), lambda i: (i, 0)),
            pl.BlockSpec((BR, din), lambda i: (i, 0)),
            _wspec((din, H)), _wspec((1, H)), _wspec((din, H)),
            _wspec((H, H)), _wspec((1, H)), _wspec((H, H)), _wspec((1, H)),
            _wspec((H, H // 2)), _wspec((1, H // 2)),
            _wspec((H // 2, OUT)), _wspec((1, OUT)),
        ],
        out_specs=pl.BlockSpec((BR, OUT), lambda i: (i, 0)),
        out_shape=jax.ShapeDtypeStruct((NP, OUT), jnp.float32),
    )(h, agg, wlt, bl, wrt, w1t, b1, w2t, b2, fc1t, fc1b, fc2t, fc2b)


def _segment_sum(h, src, dst, din):
    fn = _sc_agg_256 if din == 256 else _sc_agg_512
    S = din // 128
    zeros = jnp.zeros((RPT, 128), jnp.float32)
    out = fn(h.reshape(NP * S, 128), src, dst, zeros)
    return out.reshape(NP, din)


def kernel(x, edge_index,
           conv0_Wl, conv0_bl, conv0_Wr, mlp0_W1, mlp0_b1, mlp0_W2, mlp0_b2,
           conv1_Wl, conv1_bl, conv1_Wr, mlp1_W1, mlp1_b1, mlp1_W2, mlp1_b2,
           conv2_Wl, conv2_bl, conv2_Wr, mlp2_W1, mlp2_b1, mlp2_W2, mlp2_b2,
           fc1_W, fc1_b, fc2_W, fc2_b):
    src = edge_index[0]
    dst = edge_index[1]
    layers = [
        (conv0_Wl, conv0_bl, conv0_Wr, mlp0_W1, mlp0_b1, mlp0_W2, mlp0_b2),
        (conv1_Wl, conv1_bl, conv1_Wr, mlp1_W1, mlp1_b1, mlp1_W2, mlp1_b2),
        (conv2_Wl, conv2_bl, conv2_Wr, mlp2_W1, mlp2_b1, mlp2_W2, mlp2_b2),
    ]
    h = jnp.pad(x, ((0, NP - N), (0, 0)))
    for i in range(3):
        wl, bl, wr, w1, b1, w2, b2 = layers[i]
        args = (wl.T, bl[None, :], wr.T, w1.T, b1[None, :], w2.T, b2[None, :])
        agg = _segment_sum(h, src, dst, h.shape[1])
        if i < 2:
            h = _dense_layer(h, agg, *args)
        else:
            out = _final_layer(h, agg, *args,
                               fc1_W.T, fc1_b[None, :], fc2_W.T, fc2_b[None, :])
    return out[:N]


# speedup vs baseline: 1.0282x; 1.0282x over previous
import functools

import jax
import jax.numpy as jnp
from jax import lax
from jax.experimental import pallas as pl
from jax.experimental.pallas import tpu as pltpu
from jax.experimental.pallas import tpu_sc as plsc

N = 10000
E = 160000
NP = 10240
H = 512
OUT = 64
BR = 1024

NC = 2
NS = 16
L = 16
RC = 4000
KSUB = 128
KEPT = RC + 176
HALF = NP // NC
ACC_SUB = 10368
RPT = ACC_SUB // NS
NCH = E // RC


def _make_sc_agg(D):
    S = D // 128
    KR = KSUB // S
    chunk_rows = 2560 if D == 512 else 5120
    qpc = HALF // chunk_rows
    own = chunk_rows // NS
    real_sub = chunk_rows * S
    wpt = real_sub // NS

    mesh = plsc.VectorSubcoreMesh(core_axis_name="c", subcore_axis_name="s")

    @functools.partial(
        pl.kernel,
        out_type=jax.ShapeDtypeStruct((NP * S, 128), jnp.float32),
        mesh=mesh,
        scratch_types=[
            pltpu.VMEM((RC,), jnp.int32),
            pltpu.VMEM((RC,), jnp.int32),
            pltpu.VMEM((RC,), jnp.int32),
            pltpu.VMEM((RC,), jnp.int32),
            pltpu.VMEM((KEPT,), jnp.int32),
            pltpu.VMEM((KEPT,), jnp.int32),
            pltpu.VMEM((KSUB,), jnp.int32),
            pltpu.VMEM((KSUB,), jnp.int32),
            pltpu.VMEM((KSUB, 128), jnp.float32),
            pltpu.VMEM_SHARED((ACC_SUB, 128), jnp.float32),
            pltpu.SemaphoreType.DMA,
            pltpu.SemaphoreType.DMA,
            pltpu.SemaphoreType.DMA,
            pltpu.SemaphoreType.DMA,
            pltpu.SemaphoreType.DMA,
        ],
        compiler_params=pltpu.CompilerParams(needs_layout_passes=False),
    )
    def body(h_hbm, src_hbm, dst_hbm, zeros_hbm, out_hbm,
             raw_srcA, raw_dstA, raw_srcB, raw_dstB,
             kept_src, kept_dst, idxg, idxd, gbuf,
             acc, sem, semA, semB, semC, semD):
        c = lax.axis_index("c")
        s = lax.axis_index("s")
        lane = lax.iota(jnp.int32, L)
        dummy = jnp.full((L,), chunk_rows, jnp.int32)
        zi = jnp.zeros((L,), jnp.int32)

        def drain(nb):
            def gbody(j, carry):
                o = j * KR
                for hh in range(KR // L):
                    sv = kept_src[pl.ds(o + hh * L, L)]
                    dv = kept_dst[pl.ds(o + hh * L, L)]
                    for t in range(S):
                        idxg[pl.ds(t * KR + hh * L, L)] = sv * S + t
                        idxd[pl.ds(t * KR + hh * L, L)] = dv * S + t
                pltpu.async_copy(h_hbm.at[idxg], gbuf, sem).wait()
                pltpu.sync_copy(gbuf, acc.at[idxd], add=True)
                return carry
            lax.fori_loop(0, nb, gbody, 0)

        for qi in range(qpc):
            lo = (c * qpc + qi) * chunk_rows
            tlo = lo + s * own

            pltpu.sync_copy(zeros_hbm, acc.at[pl.ds(s * RPT, RPT)])
            plsc.subcore_barrier()

            def filt(raw_src, raw_dst, offv):
                def fbody(i, offv):
                    sv = raw_src[pl.ds(i * L, L)]
                    dv = raw_dst[pl.ds(i * L, L)]
                    m = (dv >= tlo) & (dv < tlo + own)
                    mi = m.astype(jnp.int32)
                    nav = plsc.all_reduce_population_count(m)
                    pos = offv + plsc.cumsum(mi) - 1
                    plsc.store_scatter(kept_src, [pos], sv, mask=m)
                    plsc.store_scatter(kept_dst, [pos], dv - lo, mask=m)
                    return offv + nav
                return lax.fori_loop(0, RC // L, fbody, offv)

            def drain_move(offv):
                cnt = offv[0]
                nbf = cnt // KR
                drain(nbf)
                o = nbf * KR
                for t in range(KR // L):
                    vs = kept_src[pl.ds(o + t * L, L)]
                    vd = kept_dst[pl.ds(o + t * L, L)]
                    kept_src[pl.ds(t * L, L)] = vs
                    kept_dst[pl.ds(t * L, L)] = vd
                return offv - nbf * KR

            def rpair(p, offv):
                i0 = 2 * p
                i1 = 2 * p + 1
                c0s = pltpu.async_copy(src_hbm.at[pl.ds(i0 * RC, RC)],
                                       raw_srcA, semA)
                c0d = pltpu.async_copy(dst_hbm.at[pl.ds(i0 * RC, RC)],
                                       raw_dstA, semB)
                c1s = pltpu.async_copy(src_hbm.at[pl.ds(i1 * RC, RC)],
                                       raw_srcB, semC)
                c1d = pltpu.async_copy(dst_hbm.at[pl.ds(i1 * RC, RC)],
                                       raw_dstB, semD)
                c0s.wait()
                c0d.wait()
                offv = drain_move(filt(raw_srcA, raw_dstA, offv))
                c1s.wait()
                c1d.wait()
                return drain_move(filt(raw_srcB, raw_dstB, offv))
            remv = lax.fori_loop(0, NCH // 2, rpair, zi)

            rem = remv[0]
            for t in range(KR // L):
                plsc.store_scatter(kept_src, [remv + t * L + lane], zi)
                plsc.store_scatter(kept_dst, [remv + t * L + lane], dummy)
            drain((rem + KR - 1) // KR)

            pltpu.sync_copy(
                acc.at[pl.ds(s * wpt, wpt)],
                out_hbm.at[pl.ds((c * qpc + qi) * real_sub + s * wpt, wpt)])
            plsc.subcore_barrier()

    return body


_sc_agg_256 = _make_sc_agg(256)
_sc_agg_512 = _make_sc_agg(512)


def _hr_body(h_ref, wrt, bl, out_ref):
    out_ref[...] = (
        jnp.dot(h_ref[...], wrt[...], preferred_element_type=jnp.float32)
        + bl[...]
    )


def _hr_layer(h, wrt, bl):
    din = h.shape[1]
    return pl.pallas_call(
        _hr_body,
        grid=(NP // BR,),
        in_specs=[
            pl.BlockSpec((BR, din), lambda i: (i, 0)),
            _wspec((din, H)), _wspec((1, H)),
        ],
        out_specs=pl.BlockSpec((BR, H), lambda i: (i, 0)),
        out_shape=jax.ShapeDtypeStruct((NP, H), jnp.float32),
    )(h, wrt, bl)


def _dense_body(hr_ref, agg_ref, wlt, w1t, b1, w2t, b2, out_ref):
    t = (
        jnp.dot(agg_ref[...], wlt[...], preferred_element_type=jnp.float32)
        + hr_ref[...]
    )
    h1 = jnp.maximum(
        jnp.dot(t, w1t[...], preferred_element_type=jnp.float32) + b1[...], 0.0
    )
    h2 = jnp.maximum(
        jnp.dot(h1, w2t[...], preferred_element_type=jnp.float32) + b2[...], 0.0
    )
    out_ref[...] = h2


def _final_body(hr_ref, agg_ref, wlt, w1t, b1, w2t, b2,
                fc1t, fc1b, fc2t, fc2b, out_ref):
    t = (
        jnp.dot(agg_ref[...], wlt[...], preferred_element_type=jnp.float32)
        + hr_ref[...]
    )
    h1 = jnp.maximum(
        jnp.dot(t, w1t[...], preferred_element_type=jnp.float32) + b1[...], 0.0
    )
    h2 = jnp.maximum(
        jnp.dot(h1, w2t[...], preferred_element_type=jnp.float32) + b2[...], 0.0
    )
    f1 = jnp.maximum(
        jnp.dot(h2, fc1t[...], preferred_element_type=jnp.float32) + fc1b[...], 0.0
    )
    f2 = jnp.dot(f1, fc2t[...], preferred_element_type=jnp.float32) + fc2b[...]
    out_ref[...] = 1.0 / (1.0 + jnp.exp(-f2))


def _wspec(shape):
    return pl.BlockSpec(shape, lambda i: (0, 0))


def _dense_layer(hr, agg, wlt, w1t, b1, w2t, b2):
    din = agg.shape[1]
    grid = (NP // BR,)
    return pl.pallas_call(
        _dense_body,
        grid=grid,
        in_specs=[
            pl.BlockSpec((BR, H), lambda i: (i, 0)),
            pl.BlockSpec((BR, din), lambda i: (i, 0)),
            _wspec((din, H)),
            _wspec((H, H)), _wspec((1, H)), _wspec((H, H)), _wspec((1, H)),
        ],
        out_specs=pl.BlockSpec((BR, H), lambda i: (i, 0)),
        out_shape=jax.ShapeDtypeStruct((NP, H), jnp.float32),
    )(hr, agg, wlt, w1t, b1, w2t, b2)


def _final_layer(hr, agg, wlt, w1t, b1, w2t, b2, fc1t, fc1b, fc2t, fc2b):
    din = agg.shape[1]
    grid = (NP // BR,)
    return pl.pallas_call(
        _final_body,
        grid=grid,
        in_specs=[
            pl.BlockSpec((BR, H), lambda i: (i, 0)),
            pl.BlockSpec((BR, din), lambda i: (i, 0)),
            _wspec((din, H)),
            _wspec((H, H)), _wspec((1, H)), _wspec((H, H)), _wspec((1, H)),
            _wspec((H, H // 2)), _wspec((1, H // 2)),
            _wspec((H // 2, OUT)), _wspec((1, OUT)),
        ],
        out_specs=pl.BlockSpec((BR, OUT), lambda i: (i, 0)),
        out_shape=jax.ShapeDtypeStruct((NP, OUT), jnp.float32),
    )(hr, agg, wlt, w1t, b1, w2t, b2, fc1t, fc1b, fc2t, fc2b)


def _segment_sum(h, src, dst, din):
    fn = _sc_agg_256 if din == 256 else _sc_agg_512
    S = din // 128
    zeros = jnp.zeros((RPT, 128), jnp.float32)
    out = fn(h.reshape(NP * S, 128), src, dst, zeros)
    return out.reshape(NP, din)


def kernel(x, edge_index,
           conv0_Wl, conv0_bl, conv0_Wr, mlp0_W1, mlp0_b1, mlp0_W2, mlp0_b2,
           conv1_Wl, conv1_bl, conv1_Wr, mlp1_W1, mlp1_b1, mlp1_W2, mlp1_b2,
           conv2_Wl, conv2_bl, conv2_Wr, mlp2_W1, mlp2_b1, mlp2_W2, mlp2_b2,
           fc1_W, fc1_b, fc2_W, fc2_b):
    src = edge_index[0]
    dst = edge_index[1]
    layers = [
        (conv0_Wl, conv0_bl, conv0_Wr, mlp0_W1, mlp0_b1, mlp0_W2, mlp0_b2),
        (conv1_Wl, conv1_bl, conv1_Wr, mlp1_W1, mlp1_b1, mlp1_W2, mlp1_b2),
        (conv2_Wl, conv2_bl, conv2_Wr, mlp2_W1, mlp2_b1, mlp2_W2, mlp2_b2),
    ]
    h = jnp.pad(x, ((0, NP - N), (0, 0)))
    for i in range(3):
        wl, bl, wr, w1, b1, w2, b2 = layers[i]
        args = (wl.T, w1.T, b1[None, :], w2.T, b2[None, :])
        hr = _hr_layer(h, wr.T, bl[None, :])
        agg = _segment_sum(h, src, dst, h.shape[1])
        if i < 2:
            h = _dense_layer(hr, agg, *args)
        else:
            out = _final_layer(hr, agg, *args,
                               fc1_W.T, fc1_b[None, :], fc2_W.T, fc2_b[None, :])
    return out[:N]


# speedup vs baseline: 1.0348x; 1.0064x over previous
import functools

import jax
import jax.numpy as jnp
from jax import lax
from jax.experimental import pallas as pl
from jax.experimental.pallas import tpu as pltpu
from jax.experimental.pallas import tpu_sc as plsc

N = 10000
E = 160000
NP = 10240
H = 512
OUT = 64
BR = 1024

NC = 2
NS = 16
L = 16
RC = 4000
KSUB = 128
KEPT = RC + 176
HALF = NP // NC
ACC_SUB = 10368
RPT = ACC_SUB // NS
NCH = E // RC


def _make_sc_agg(D):
    S = D // 128
    KR = KSUB // S
    chunk_rows = 2560 if D == 512 else 5120
    qpc = HALF // chunk_rows
    own = chunk_rows // NS
    real_sub = chunk_rows * S
    wpt = real_sub // NS

    mesh = plsc.VectorSubcoreMesh(core_axis_name="c", subcore_axis_name="s")

    @functools.partial(
        pl.kernel,
        out_type=jax.ShapeDtypeStruct((NP * S, 128), jnp.float32),
        mesh=mesh,
        scratch_types=[
            pltpu.VMEM((RC,), jnp.int32),
            pltpu.VMEM((RC,), jnp.int32),
            pltpu.VMEM((RC,), jnp.int32),
            pltpu.VMEM((RC,), jnp.int32),
            pltpu.VMEM((KEPT,), jnp.int32),
            pltpu.VMEM((KEPT,), jnp.int32),
            pltpu.VMEM((KSUB,), jnp.int32),
            pltpu.VMEM((KSUB,), jnp.int32),
            pltpu.VMEM((KSUB, 128), jnp.float32),
            pltpu.VMEM_SHARED((ACC_SUB, 128), jnp.float32),
            pltpu.SemaphoreType.DMA,
            pltpu.SemaphoreType.DMA,
            pltpu.SemaphoreType.DMA,
            pltpu.SemaphoreType.DMA,
            pltpu.SemaphoreType.DMA,
        ],
        compiler_params=pltpu.CompilerParams(needs_layout_passes=False),
    )
    def body(h_hbm, src_hbm, dst_hbm, zeros_hbm, out_hbm,
             raw_srcA, raw_dstA, raw_srcB, raw_dstB,
             kept_src, kept_dst, idxg, idxd, gbuf,
             acc, sem, semA, semB, semC, semD):
        c = lax.axis_index("c")
        s = lax.axis_index("s")
        lane = lax.iota(jnp.int32, L)
        dummy = jnp.full((L,), chunk_rows, jnp.int32)
        zi = jnp.zeros((L,), jnp.int32)

        def drain(nb):
            def gbody(j, carry):
                o = j * KR
                for hh in range(KR // L):
                    sv = kept_src[pl.ds(o + hh * L, L)]
                    dv = kept_dst[pl.ds(o + hh * L, L)]
                    for t in range(S):
                        idxg[pl.ds(t * KR + hh * L, L)] = sv * S + t
                        idxd[pl.ds(t * KR + hh * L, L)] = dv * S + t
                pltpu.async_copy(h_hbm.at[idxg], gbuf, sem).wait()
                pltpu.sync_copy(gbuf, acc.at[idxd], add=True)
                return carry
            lax.fori_loop(0, nb, gbody, 0)

        for qi in range(qpc):
            lo = (c * qpc + qi) * chunk_rows
            tlo = lo + s * own

            pltpu.sync_copy(zeros_hbm, acc.at[pl.ds(s * RPT, RPT)])
            plsc.subcore_barrier()

            def filt(raw_src, raw_dst, offv):
                def fbody(i, offv):
                    sv = raw_src[pl.ds(i * L, L)]
                    dv = raw_dst[pl.ds(i * L, L)]
                    m = (dv >= tlo) & (dv < tlo + own)
                    mi = m.astype(jnp.int32)
                    nav = plsc.all_reduce_population_count(m)
                    pos = offv + plsc.cumsum(mi) - 1
                    plsc.store_scatter(kept_src, [pos], sv, mask=m)
                    plsc.store_scatter(kept_dst, [pos], dv - lo, mask=m)
                    return offv + nav
                return lax.fori_loop(0, RC // L, fbody, offv)

            def drain_move(offv):
                cnt = offv[0]
                nbf = cnt // KR
                drain(nbf)
                o = nbf * KR
                for t in range(KR // L):
                    vs = kept_src[pl.ds(o + t * L, L)]
                    vd = kept_dst[pl.ds(o + t * L, L)]
                    kept_src[pl.ds(t * L, L)] = vs
                    kept_dst[pl.ds(t * L, L)] = vd
                return offv - nbf * KR

            def rpair(p, offv):
                i0 = 2 * p
                i1 = 2 * p + 1
                c0s = pltpu.async_copy(src_hbm.at[pl.ds(i0 * RC, RC)],
                                       raw_srcA, semA)
                c0d = pltpu.async_copy(dst_hbm.at[pl.ds(i0 * RC, RC)],
                                       raw_dstA, semB)
                c1s = pltpu.async_copy(src_hbm.at[pl.ds(i1 * RC, RC)],
                                       raw_srcB, semC)
                c1d = pltpu.async_copy(dst_hbm.at[pl.ds(i1 * RC, RC)],
                                       raw_dstB, semD)
                c0s.wait()
                c0d.wait()
                offv = drain_move(filt(raw_srcA, raw_dstA, offv))
                c1s.wait()
                c1d.wait()
                return drain_move(filt(raw_srcB, raw_dstB, offv))
            remv = lax.fori_loop(0, NCH // 2, rpair, zi)

            rem = remv[0]
            for t in range(KR // L):
                plsc.store_scatter(kept_src, [remv + t * L + lane], zi)
                plsc.store_scatter(kept_dst, [remv + t * L + lane], dummy)
            drain((rem + KR - 1) // KR)

            pltpu.sync_copy(
                acc.at[pl.ds(s * wpt, wpt)],
                out_hbm.at[pl.ds((c * qpc + qi) * real_sub + s * wpt, wpt)])
            plsc.subcore_barrier()

    return body


_sc_agg_256 = _make_sc_agg(256)
_sc_agg_512 = _make_sc_agg(512)


def _dense_body(h_ref, agg_ref, wlt, bl, wrt, w1t, b1, w2t, b2, out_ref):
    t = (
        jnp.dot(agg_ref[...], wlt[...], preferred_element_type=jnp.float32)
        + bl[...]
        + jnp.dot(h_ref[...], wrt[...], preferred_element_type=jnp.float32)
    )
    h1 = jnp.maximum(
        jnp.dot(t, w1t[...], preferred_element_type=jnp.float32) + b1[...], 0.0
    )
    h2 = jnp.maximum(
        jnp.dot(h1, w2t[...], preferred_element_type=jnp.float32) + b2[...], 0.0
    )
    out_ref[...] = h2


def _final_body(h_ref, agg_ref, wlt, bl, wrt, w1t, b1, w2t, b2,
                fc1t, fc1b, fc2t, fc2b, out_ref):
    t = (
        jnp.dot(agg_ref[...], wlt[...], preferred_element_type=jnp.float32)
        + bl[...]
        + jnp.dot(h_ref[...], wrt[...], preferred_element_type=jnp.float32)
    )
    h1 = jnp.maximum(
        jnp.dot(t, w1t[...], preferred_element_type=jnp.float32) + b1[...], 0.0
    )
    h2 = jnp.maximum(
        jnp.dot(h1, w2t[...], preferred_element_type=jnp.float32) + b2[...], 0.0
    )
    f1 = jnp.maximum(
        jnp.dot(h2, fc1t[...], preferred_element_type=jnp.float32) + fc1b[...], 0.0
    )
    f2 = jnp.dot(f1, fc2t[...], preferred_element_type=jnp.float32) + fc2b[...]
    out_ref[...] = 1.0 / (1.0 + jnp.exp(-f2))


def _wspec(shape):
    return pl.BlockSpec(shape, lambda i: (0, 0))


def _dense_layer(h, agg, wlt, bl, wrt, w1t, b1, w2t, b2):
    din = h.shape[1]
    grid = (NP // BR,)
    return pl.pallas_call(
        _dense_body,
        grid=grid,
        in_specs=[
            pl.BlockSpec((BR, din), lambda i: (i, 0)),
            pl.BlockSpec((BR, din), lambda i: (i, 0)),
            _wspec((din, H)), _wspec((1, H)), _wspec((din, H)),
            _wspec((H, H)), _wspec((1, H)), _wspec((H, H)), _wspec((1, H)),
        ],
        out_specs=pl.BlockSpec((BR, H), lambda i: (i, 0)),
        out_shape=jax.ShapeDtypeStruct((NP, H), jnp.float32),
    )(h, agg, wlt, bl, wrt, w1t, b1, w2t, b2)


def _final_layer(h, agg, wlt, bl, wrt, w1t, b1, w2t, b2,
                 fc1t, fc1b, fc2t, fc2b):
    din = h.shape[1]
    grid = (NP // BR,)
    return pl.pallas_call(
        _final_body,
        grid=grid,
        in_specs=[
            pl.BlockSpec((BR, din), lambda i: (i, 0)),
            pl.BlockSpec((BR, din), lambda i: (i, 0)),
            _wspec((din, H)), _wspec((1, H)), _wspec((din, H)),
            _wspec((H, H)), _wspec((1, H)), _wspec((H, H)), _wspec((1, H)),
            _wspec((H, H // 2)), _wspec((1, H // 2)),
            _wspec((H // 2, OUT)), _wspec((1, OUT)),
        ],
        out_specs=pl.BlockSpec((BR, OUT), lambda i: (i, 0)),
        out_shape=jax.ShapeDtypeStruct((NP, OUT), jnp.float32),
    )(h, agg, wlt, bl, wrt, w1t, b1, w2t, b2, fc1t, fc1b, fc2t, fc2b)


def _segment_sum(h, src, dst, din):
    fn = _sc_agg_256 if din == 256 else _sc_agg_512
    S = din // 128
    zeros = jnp.zeros((RPT, 128), jnp.float32)
    out = fn(h.reshape(NP * S, 128), src, dst, zeros)
    return out.reshape(NP, din)


def kernel(x, edge_index,
           conv0_Wl, conv0_bl, conv0_Wr, mlp0_W1, mlp0_b1, mlp0_W2, mlp0_b2,
           conv1_Wl, conv1_bl, conv1_Wr, mlp1_W1, mlp1_b1, mlp1_W2, mlp1_b2,
           conv2_Wl, conv2_bl, conv2_Wr, mlp2_W1, mlp2_b1, mlp2_W2, mlp2_b2,
           fc1_W, fc1_b, fc2_W, fc2_b):
    src = edge_index[0]
    dst = edge_index[1]
    layers = [
        (conv0_Wl, conv0_bl, conv0_Wr, mlp0_W1, mlp0_b1, mlp0_W2, mlp0_b2),
        (conv1_Wl, conv1_bl, conv1_Wr, mlp1_W1, mlp1_b1, mlp1_W2, mlp1_b2),
        (conv2_Wl, conv2_bl, conv2_Wr, mlp2_W1, mlp2_b1, mlp2_W2, mlp2_b2),
    ]
    h = jnp.pad(x, ((0, NP - N), (0, 0)))
    for i in range(3):
        wl, bl, wr, w1, b1, w2, b2 = layers[i]
        args = (wl.T, bl[None, :], wr.T, w1.T, b1[None, :], w2.T, b2[None, :])
        agg = _segment_sum(h, src, dst, h.shape[1])
        if i < 2:
            h = _dense_layer(h, agg, *args)
        else:
            out = _final_layer(h, agg, *args,
                               fc1_W.T, fc1_b[None, :], fc2_W.T, fc2_b[None, :])
    return out[:N]
